# Initial kernel scaffold; baseline (speedup 1.0000x reference)
import jax
import jax.numpy as jnp
from jax.experimental import pallas as pl


def kernel(candidate_images, selection_scores, mask, partial_image):
    raise NotImplementedError("write your pallas kernel here")


# speedup vs baseline: 2.9936x; 2.9936x over previous
import jax
import jax.numpy as jnp
from jax.experimental import pallas as pl

B, K, C, H, W = 4, 16, 3, 384, 384
HB = 64


def _selector_block(scores_ref, cand_ref, mask_ref, partial_ref,
                    final_ref, weights_ref):
    scores = scores_ref[0]
    best = jnp.argmax(scores, axis=0).astype(jnp.int32)

    vis = mask_ref[0, 0]
    fill = 1.0 - vis

    sel = [None, None, None]
    for k in range(K):
        onehot = (best == k)
        weights_ref[0, k] = onehot.astype(jnp.float32)
        for c in range(C):
            pix = jnp.where(onehot, cand_ref[0, k, c], 0.0)
            sel[c] = pix if sel[c] is None else sel[c] + pix

    for c in range(C):
        final_ref[0, c] = partial_ref[0, c] * vis + sel[c] * fill


def kernel(candidate_images, selection_scores, mask, partial_image):
    grid = (B, H // HB)
    final_image, selection_weights = pl.pallas_call(
        _selector_block,
        grid=grid,
        in_specs=[
            pl.BlockSpec((1, K, HB, W), lambda b, h: (b, 0, h, 0)),
            pl.BlockSpec((1, K, C, HB, W), lambda b, h: (b, 0, 0, h, 0)),
            pl.BlockSpec((1, 1, HB, W), lambda b, h: (b, 0, h, 0)),
            pl.BlockSpec((1, C, HB, W), lambda b, h: (b, 0, h, 0)),
        ],
        out_specs=[
            pl.BlockSpec((1, C, HB, W), lambda b, h: (b, 0, h, 0)),
            pl.BlockSpec((1, K, HB, W), lambda b, h: (b, 0, h, 0)),
        ],
        out_shape=[
            jax.ShapeDtypeStruct((B, C, H, W), jnp.float32),
            jax.ShapeDtypeStruct((B, K, H, W), jnp.float32),
        ],
    )(selection_scores, candidate_images, mask, partial_image)
    return (final_image, selection_weights)


# speedup vs baseline: 2.9939x; 1.0001x over previous
import jax
import jax.numpy as jnp
from jax.experimental import pallas as pl

B, K, C, H, W = 4, 16, 3, 384, 384
HB = 128


def _selector_block(scores_ref, cand_ref, mask_ref, partial_ref,
                    final_ref, weights_ref):
    scores = scores_ref[0]
    best = jnp.argmax(scores, axis=0).astype(jnp.int32)

    vis = mask_ref[0, 0]
    fill = 1.0 - vis

    sel = [None, None, None]
    for k in range(K):
        onehot = (best == k)
        weights_ref[0, k] = onehot.astype(jnp.float32)
        for c in range(C):
            pix = jnp.where(onehot, cand_ref[0, k, c], 0.0)
            sel[c] = pix if sel[c] is None else sel[c] + pix

    for c in range(C):
        final_ref[0, c] = partial_ref[0, c] * vis + sel[c] * fill


def kernel(candidate_images, selection_scores, mask, partial_image):
    grid = (B, H // HB)
    final_image, selection_weights = pl.pallas_call(
        _selector_block,
        grid=grid,
        in_specs=[
            pl.BlockSpec((1, K, HB, W), lambda b, h: (b, 0, h, 0)),
            pl.BlockSpec((1, K, C, HB, W), lambda b, h: (b, 0, 0, h, 0)),
            pl.BlockSpec((1, 1, HB, W), lambda b, h: (b, 0, h, 0)),
            pl.BlockSpec((1, C, HB, W), lambda b, h: (b, 0, h, 0)),
        ],
        out_specs=[
            pl.BlockSpec((1, C, HB, W), lambda b, h: (b, 0, h, 0)),
            pl.BlockSpec((1, K, HB, W), lambda b, h: (b, 0, h, 0)),
        ],
        out_shape=[
            jax.ShapeDtypeStruct((B, C, H, W), jnp.float32),
            jax.ShapeDtypeStruct((B, K, H, W), jnp.float32),
        ],
    )(selection_scores, candidate_images, mask, partial_image)
    return (final_image, selection_weights)
